# baseline (speedup 1.0000x reference)
import functools
import jax
import jax.numpy as jnp
from jax import lax
from jax.experimental import pallas as pl
from jax.experimental.pallas import tpu as pltpu
from jax.experimental.pallas import tpu_sc as plsc

NEG = jnp.finfo(jnp.float32).min

_N = 8192
_NW = 32
_RPW = _N // _NW
_RC = 4
_NCH = _RPW // _RC
_LANES = 16
_KV = _N // _LANES


def _sc_sum_body(adj_hbm, f1_hbm, out_hbm, f1_v, buf0, buf1, out_v, acc_buf,
                 sem0, sem1):
    wid = lax.axis_index("s") * 2 + lax.axis_index("c")
    base = wid * _RPW
    pltpu.sync_copy(f1_hbm, f1_v)
    bufs = (buf0, buf1)
    sems = (sem0, sem1)
    lanes = lax.iota(jnp.int32, _LANES)

    pltpu.async_copy(adj_hbm.at[pl.ds(base, _RC)], buf0, sem0)

    def group_body(g, _):
        for cc in range(_LANES // _RC):
            c = g * (_LANES // _RC) + cc
            p = cc % 2
            buf = bufs[p]
            row0 = base + c * _RC
            pltpu.make_async_copy(adj_hbm.at[pl.ds(row0, _RC)], buf, sems[p]).wait()

            @pl.when(c + 1 < _NCH)
            def _prefetch():
                pltpu.async_copy(
                    adj_hbm.at[pl.ds(row0 + _RC, _RC)], bufs[1 - p], sems[1 - p])

            def kbody(k, accs):
                off = k * _LANES
                f = f1_v[pl.ds(off, _LANES)]
                return tuple(
                    accs[r] + jnp.where(buf[r, pl.ds(off, _LANES)] != 0, f, 0.0)
                    for r in range(_RC))

            accs = lax.fori_loop(
                0, _KV, kbody, tuple(jnp.zeros((_LANES,), jnp.float32)
                                     for _ in range(_RC)))
            for r in range(_RC):
                acc_buf[cc * _RC + r, :] = accs[r]
        res = jnp.zeros((_LANES,), jnp.float32)
        for t in range(_LANES):
            col = jnp.full((_LANES,), t, jnp.int32)
            res = res + plsc.load_gather(acc_buf, [lanes, col])
        out_v[pl.ds(g * _LANES, _LANES)] = res
        return 0

    lax.fori_loop(0, _RPW // _LANES, group_body, 0)
    pltpu.sync_copy(out_v, out_hbm.at[pl.ds(base, _RPW)])


def _sc_scal2(adj_1to2, f1_row):
    mesh = plsc.VectorSubcoreMesh(core_axis_name="c", subcore_axis_name="s")
    return pl.kernel(
        _sc_sum_body,
        out_type=jax.ShapeDtypeStruct((_N,), jnp.float32),
        mesh=mesh,
        compiler_params=pltpu.CompilerParams(needs_layout_passes=False),
        scratch_types=[
            pltpu.VMEM((_N,), jnp.float32),
            pltpu.VMEM((_RC, _N), jnp.int32),
            pltpu.VMEM((_RC, _N), jnp.int32),
            pltpu.VMEM((_RPW,), jnp.float32),
            pltpu.VMEM((_LANES, _LANES), jnp.float32),
            pltpu.SemaphoreType.DMA,
            pltpu.SemaphoreType.DMA,
        ],
    )(adj_1to2, f1_row)


def _tc_max_body(adj21, f2, x1, w1s, w1n, out1, m_acc, h_acc, *, n_col_blocks):
    c = pl.program_id(1)
    a21 = adj21[...]
    vals = jnp.where(a21 != 0, f2[...], NEG)
    m = jnp.max(vals, axis=1, keepdims=True)
    h = jnp.max(a21, axis=1, keepdims=True)

    @pl.when(c == 0)
    def _init():
        m_acc[...] = m
        h_acc[...] = h

    @pl.when(c > 0)
    def _accum():
        m_acc[...] = jnp.maximum(m_acc[...], m)
        h_acc[...] = jnp.maximum(h_acc[...], h)

    @pl.when(c == n_col_blocks - 1)
    def _finalize():
        scal1 = jnp.where(h_acc[...] > 0, m_acc[...], 0.0)
        wsum1 = jnp.sum(w1n[...], axis=1)
        o1 = jnp.dot(x1[...], w1s[...].T, preferred_element_type=jnp.float32)
        out1[...] = jnp.maximum(o1 + scal1 * wsum1[None, :], 0.0)


def _tc_out2_body(scal2, x2, w2s, w2n, out2):
    wsum2 = jnp.sum(w2n[...], axis=1)
    o2 = jnp.dot(x2[...], w2s[...].T, preferred_element_type=jnp.float32)
    out2[...] = jnp.maximum(o2 + scal2[...] * wsum2[None, :], 0.0)


def kernel(node_feats1, node_feats2, adj_1to2, adj_2to1,
           W1_self, W1_neigh, W2_self, W2_neigh):
    n1, d_in = node_feats1.shape
    n2, _ = node_feats2.shape
    d_out = W1_self.shape[0]

    f1_row = node_feats1[:, 0]
    f2_row = node_feats2[:, 0].reshape(1, n2)

    scal2 = _sc_scal2(adj_1to2, f1_row)

    br = 256
    bc = 2048
    nr = n1 // br
    nc = n2 // bc
    out1 = pl.pallas_call(
        functools.partial(_tc_max_body, n_col_blocks=nc),
        grid=(nr, nc),
        in_specs=[
            pl.BlockSpec((br, bc), lambda r, c: (r, c)),
            pl.BlockSpec((1, bc), lambda r, c: (0, c)),
            pl.BlockSpec((br, d_in), lambda r, c: (r, 0)),
            pl.BlockSpec((d_out, d_in), lambda r, c: (0, 0)),
            pl.BlockSpec((d_out, d_in), lambda r, c: (0, 0)),
        ],
        out_specs=pl.BlockSpec((br, d_out), lambda r, c: (r, 0)),
        out_shape=jax.ShapeDtypeStruct((n1, d_out), jnp.float32),
        scratch_shapes=[
            pltpu.VMEM((br, 1), jnp.float32),
            pltpu.VMEM((br, 1), jnp.int32),
        ],
        compiler_params=pltpu.CompilerParams(
            dimension_semantics=("parallel", "arbitrary"),
        ),
    )(adj_2to1, f2_row, node_feats1, W1_self, W1_neigh)

    out2 = pl.pallas_call(
        _tc_out2_body,
        grid=(n2 // br,),
        in_specs=[
            pl.BlockSpec((br, 1), lambda r: (r, 0)),
            pl.BlockSpec((br, d_in), lambda r: (r, 0)),
            pl.BlockSpec((d_out, d_in), lambda r: (0, 0)),
            pl.BlockSpec((d_out, d_in), lambda r: (0, 0)),
        ],
        out_specs=pl.BlockSpec((br, d_out), lambda r: (r, 0)),
        out_shape=jax.ShapeDtypeStruct((n2, d_out), jnp.float32),
        compiler_params=pltpu.CompilerParams(
            dimension_semantics=("arbitrary",),
        ),
    )(scal2.reshape(n2, 1), node_feats2, W2_self, W2_neigh)

    return out1, out2
